# baseline (speedup 1.0000x reference)
import dataclasses
import functools

import numpy as np
import jax
from jax import lax
import jax.numpy as jnp
from jax.experimental import pallas as pl
from jax.experimental.pallas import tpu as pltpu
from jax.experimental.pallas import tpu_sc as plsc

NF = 26
FD = 4000
VOCAB = NF * FD
D = 16
NPAIR = NF * (NF - 1) // 2
B = 4096
H = 400
FFM_OUT = NPAIR * D
EPS = 1e-5

NC, NS = 2, 16
NW = NC * NS
BW = B // NW

VROW = 104448

_II, _JJ = np.triu_indices(NF, k=1)
_PF = np.zeros(2 * NPAIR, np.int32)
_PC = np.zeros(2 * NPAIR, np.int32)
_PF[0::2] = _II
_PC[0::2] = _JJ * VROW + _II * FD
_PF[1::2] = _JJ
_PC[1::2] = _II * VROW + _JJ * FD
NL = 656
_PFP = np.zeros(NL, np.int32)
_PCP = np.zeros(NL, np.int32)
_PFP[: 2 * NPAIR] = _PF
_PCP[: 2 * NPAIR] = _PC

_CHUNKS = [(0, 128), (128, 128), (256, 128), (384, 128), (512, 128), (640, 16)]

_PAIR_UNROLL = 13

KT = 41
NPS = 328
NSC = 336
OROWS = (B // 8) * KT * 64
_TOFF = np.zeros(NSC, np.int32)
_p = np.arange(NPS, dtype=np.int64)
_TOFF[:NPS] = ((_p >> 3) * 64 + (_p & 7)).astype(np.int32)
_TOFF[NPS:] = _TOFF[326]
_SCHUNKS = [(0, 128), (128, 128), (256, 80)]


def _sc_body(xT, tabs, linw16, pf_h, pc_h, toff_h,
             ocross, stats, linsum,
             pf_v, pc_v, toff_v, xblk, lidx0, lidx1, lidx2, lidx3,
             rows0, rows1, rows2, rows3,
             prod0, prod1, s0a, s1a, s2a, s0b, s1b, s2b,
             li_idx, li_lane, li_rows, li_acc, st_v,
             sem_g0, sem_g1, sem_g2, sem_g3, sem_w0, sem_w1, sem_s):
    c = lax.axis_index("c")
    s = lax.axis_index("s")
    wid = s * NC + c
    base = wid * BW

    pltpu.sync_copy(pf_h, pf_v)
    pltpu.sync_copy(pc_h, pc_v)
    pltpu.sync_copy(toff_h, toff_v)
    pltpu.sync_copy(xT.at[:, pl.ds(base, BW)], xblk)
    iota16 = lax.iota(jnp.int32, 16)

    for i in range(NF):
        for g in range(8):
            off = i * BW + g * 16
            v = xblk[i, pl.ds(g * 16, 16)]
            li_idx[pl.ds(off, 16)] = 250 * i + (v >> 4)
            li_lane[pl.ds(off, 16)] = v & 15
    for i in range(NF):
        pltpu.async_copy(linw16.at[li_idx.at[pl.ds(i * BW, BW)]],
                         li_rows.at[pl.ds(i * BW, BW)], sem_s)
    for i in range(NF):
        pltpu.make_async_copy(linw16.at[li_idx.at[pl.ds(i * BW, BW)]],
                              li_rows.at[pl.ds(i * BW, BW)], sem_s).wait()
    for g in range(8):
        acc = jnp.zeros((16,), jnp.float32)
        for i in range(NF):
            off = i * BW + g * 16
            rvec = off + iota16
            lvec = li_lane[pl.ds(off, 16)]
            acc = acc + plsc.load_gather(li_rows, [rvec, lvec])
        li_acc[pl.ds(g * 16, 16)] = acc
    pltpu.sync_copy(li_acc, linsum.at[pl.ds(base, BW)])

    st_v[0] = jnp.zeros((16,), jnp.float32)
    st_v[1] = jnp.zeros((16,), jnp.float32)

    def build_lidx(b, lidx_ref):
        bsplat = jnp.full((16,), b, jnp.int32)
        for grp in range(NL // 16):
            pfv = pf_v[pl.ds(grp * 16, 16)]
            pcv = pc_v[pl.ds(grp * 16, 16)]
            xv = plsc.load_gather(xblk, [pfv, bsplat])
            lidx_ref[pl.ds(grp * 16, 16)] = xv + pcv

    def fire_gather(lidx_ref, rows_ref, sem):
        for (o, n) in _CHUNKS:
            pltpu.async_copy(tabs.at[lidx_ref.at[pl.ds(o, n)]],
                             rows_ref.at[pl.ds(o, n)], sem)

    def drain_gather(lidx_ref, rows_ref, sem):
        for (o, n) in _CHUNKS:
            pltpu.make_async_copy(tabs.at[lidx_ref.at[pl.ds(o, n)]],
                                  rows_ref.at[pl.ds(o, n)], sem).wait()

    def compute_products(rows_ref, prod_ref):
        def chunk(t, carry):
            sa, qa = carry
            for u in range(_PAIR_UNROLL):
                p = t * _PAIR_UNROLL + u
                va = rows_ref[2 * p]
                vb = rows_ref[2 * p + 1]
                pr = va * vb
                prod_ref[p] = pr
                sa = sa + pr
                qa = qa + pr * pr
            return sa, qa
        sa, qa = lax.fori_loop(
            0, NPAIR // _PAIR_UNROLL, chunk,
            (jnp.zeros((16,), jnp.float32), jnp.zeros((16,), jnp.float32)))
        st_v[0] = st_v[0] + sa
        st_v[1] = st_v[1] + qa

    def build_sidx(b, srefs):
        b_abs = base + b
        cb = (b_abs >> 3) * (KT * 64) + (b_abs & 7) * 8
        for grp in range(NSC // 16):
            tv = toff_v[pl.ds(grp * 16, 16)]
            val = tv + cb
            if grp < 8:
                srefs[0][pl.ds(grp * 16, 16)] = val
            elif grp < 16:
                srefs[1][pl.ds((grp - 8) * 16, 16)] = val
            else:
                srefs[2][pl.ds((grp - 16) * 16, 16)] = val

    def fire_scatter(prod_ref, srefs, sem):
        for (o, n), sref in zip(_SCHUNKS, srefs):
            pltpu.async_copy(prod_ref.at[pl.ds(o, n)], ocross.at[sref], sem)

    def drain_scatter(prod_ref, srefs, sem):
        for (o, n), sref in zip(_SCHUNKS, srefs):
            pltpu.make_async_copy(prod_ref.at[pl.ds(o, n)], ocross.at[sref],
                                  sem).wait()

    for r in range(NPAIR, NSC):
        prod0[r] = jnp.zeros((16,), jnp.float32)
        prod1[r] = jnp.zeros((16,), jnp.float32)

    gbufs = [(lidx0, rows0, sem_g0), (lidx1, rows1, sem_g1),
             (lidx2, rows2, sem_g2), (lidx3, rows3, sem_g3)]
    wbufs = [(prod0, (s0a, s1a, s2a), sem_w0), (prod1, (s0b, s1b, s2b), sem_w1)]
    for k in range(4):
        build_lidx(k, gbufs[k][0])
        fire_gather(*gbufs[k])

    def phase(b, gbuf, wbuf):
        lidx_ref, rows_ref, sem_g = gbuf
        prod_ref, srefs, sem_w = wbuf
        drain_gather(lidx_ref, rows_ref, sem_g)

        @pl.when(b >= 2)
        def _():
            drain_scatter(prod_ref, srefs, sem_w)

        build_sidx(b, srefs)
        compute_products(rows_ref, prod_ref)
        fire_scatter(prod_ref, srefs, sem_w)

        @pl.when(b + 4 < BW)
        def _():
            build_lidx(b + 4, lidx_ref)
            fire_gather(lidx_ref, rows_ref, sem_g)

    @pl.loop(0, BW, step=4)
    def _(b):
        for ph in range(4):
            phase(b + ph, gbufs[ph], wbufs[ph & 1])

    drain_scatter(*wbufs[0])
    drain_scatter(*wbufs[1])
    pltpu.sync_copy(st_v, stats.at[wid])


def _sc_call(xT, tabs, linw16, pf, pc, toff):
    mesh = plsc.VectorSubcoreMesh(core_axis_name="c", subcore_axis_name="s")
    cp = pltpu.CompilerParams()
    for fld, val in (("needs_layout_passes", False),
                     ("use_tc_tiling_on_sc", False)):
        if fld in pltpu.CompilerParams.__dataclass_fields__:
            cp = dataclasses.replace(cp, **{fld: val})
    kern = pl.kernel(
        _sc_body,
        compiler_params=cp,
        out_type=(
            jax.ShapeDtypeStruct((OROWS, D), jnp.float32),
            jax.ShapeDtypeStruct((NW, 2, D), jnp.float32),
            jax.ShapeDtypeStruct((B,), jnp.float32),
        ),
        mesh=mesh,
        scratch_types=[
            pltpu.VMEM((NL,), jnp.int32),
            pltpu.VMEM((NL,), jnp.int32),
            pltpu.VMEM((NSC,), jnp.int32),
            pltpu.VMEM((NF, BW), jnp.int32),
            pltpu.VMEM((NL,), jnp.int32),
            pltpu.VMEM((NL,), jnp.int32),
            pltpu.VMEM((NL,), jnp.int32),
            pltpu.VMEM((NL,), jnp.int32),
            pltpu.VMEM((NL, D), jnp.float32),
            pltpu.VMEM((NL, D), jnp.float32),
            pltpu.VMEM((NL, D), jnp.float32),
            pltpu.VMEM((NL, D), jnp.float32),
            pltpu.VMEM((NSC, D), jnp.float32),
            pltpu.VMEM((NSC, D), jnp.float32),
            pltpu.VMEM((128,), jnp.int32),
            pltpu.VMEM((128,), jnp.int32),
            pltpu.VMEM((80,), jnp.int32),
            pltpu.VMEM((128,), jnp.int32),
            pltpu.VMEM((128,), jnp.int32),
            pltpu.VMEM((80,), jnp.int32),
            pltpu.VMEM((NF * BW,), jnp.int32),
            pltpu.VMEM((NF * BW,), jnp.int32),
            pltpu.VMEM((NF * BW, D), jnp.float32),
            pltpu.VMEM((BW,), jnp.float32),
            pltpu.VMEM((2, D), jnp.float32),
            pltpu.SemaphoreType.DMA,
            pltpu.SemaphoreType.DMA,
            pltpu.SemaphoreType.DMA,
            pltpu.SemaphoreType.DMA,
            pltpu.SemaphoreType.DMA,
            pltpu.SemaphoreType.DMA,
            pltpu.SemaphoreType.DMA,
        ],
    )
    return kern(xT, tabs, linw16, pf, pc, toff)


_GB = 256


VPAD = VROW
RPF = VPAD * D // 1024
_VC = 13056


def _fmt_tables(tabsT):
    def body(t_ref, o_ref):
        tr = jnp.transpose(t_ref[0], (1, 0))
        tr3 = tr.reshape(_VC // 8, 8, 16)
        out2 = jnp.concatenate([tr3[:, v0, :] for v0 in range(8)], axis=1)
        o_ref[...] = out2.reshape(_VC // 64, 8, 128)

    return pl.pallas_call(
        body,
        grid=(NF, VPAD // _VC),
        in_specs=[pl.BlockSpec((1, D, _VC), lambda f, c: (f, 0, c))],
        out_specs=pl.BlockSpec((_VC // 64, 8, 128),
                               lambda f, c: (f * (VPAD // _VC) + c, 0, 0)),
        out_shape=jax.ShapeDtypeStruct((NF * RPF, 8, 128), jnp.float32),
    )(tabsT)


def _mlp1(o4, a0row, W1p):
    def body(o_ref, a_ref, w_ref, z_ref, st_ref):
        i = pl.program_id(0)
        z = jnp.zeros((_GB, H), jnp.float32)
        for kt in range(KT):
            piece = o_ref[:, kt].reshape(_GB, 128)
            piece = piece * a_ref[0:1, kt * 128:(kt + 1) * 128]
            z = z + jnp.dot(piece, w_ref[kt * 128:(kt + 1) * 128, :],
                            preferred_element_type=jnp.float32)
        z_ref[...] = z

        @pl.when(i == 0)
        def _():
            st_ref[...] = jnp.zeros_like(st_ref)

        st_ref[0:1, :] += jnp.sum(z, axis=0, keepdims=True)
        st_ref[1:2, :] += jnp.sum(z * z, axis=0, keepdims=True)

    return pl.pallas_call(
        body,
        grid=(B // _GB,),
        in_specs=[pl.BlockSpec((_GB // 8, KT, 8, 128), lambda i: (i, 0, 0, 0)),
                  pl.BlockSpec((1, KT * 128), lambda i: (0, 0)),
                  pl.BlockSpec((KT * 128, H), lambda i: (0, 0))],
        out_specs=[pl.BlockSpec((_GB, H), lambda i: (i, 0)),
                   pl.BlockSpec((2, H), lambda i: (0, 0))],
        out_shape=[jax.ShapeDtypeStruct((B, H), jnp.float32),
                   jax.ShapeDtypeStruct((2, H), jnp.float32)],
    )(o4, a0row, W1p)


def _mlp2(z1, a1row, c1row, W2):
    def body(z_ref, a_ref, c_ref, w_ref, z2_ref, st_ref):
        i = pl.program_id(0)
        h = jnp.maximum(z_ref[...] * a_ref[...] + c_ref[...], 0.0)
        z2 = jnp.dot(h, w_ref[...], preferred_element_type=jnp.float32)
        z2_ref[...] = z2

        @pl.when(i == 0)
        def _():
            st_ref[...] = jnp.zeros_like(st_ref)

        st_ref[0:1, :] += jnp.sum(z2, axis=0, keepdims=True)
        st_ref[1:2, :] += jnp.sum(z2 * z2, axis=0, keepdims=True)

    return pl.pallas_call(
        body,
        grid=(B // _GB,),
        in_specs=[pl.BlockSpec((_GB, H), lambda i: (i, 0)),
                  pl.BlockSpec((1, H), lambda i: (0, 0)),
                  pl.BlockSpec((1, H), lambda i: (0, 0)),
                  pl.BlockSpec((H, H), lambda i: (0, 0))],
        out_specs=[pl.BlockSpec((_GB, H), lambda i: (i, 0)),
                   pl.BlockSpec((2, H), lambda i: (0, 0))],
        out_shape=[jax.ShapeDtypeStruct((B, H), jnp.float32),
                   jax.ShapeDtypeStruct((2, H), jnp.float32)],
    )(z1, a1row, c1row, W2)


def _mlp3(z2, a2row, c2row, W3, linf):
    def body(z_ref, a_ref, c_ref, w_ref, l_ref, o_ref):
        h = jnp.maximum(z_ref[...] * a_ref[...] + c_ref[...], 0.0)
        o = jnp.dot(h, w_ref[...], preferred_element_type=jnp.float32)
        o_ref[...] = jax.nn.sigmoid(o + l_ref[...])

    return pl.pallas_call(
        body,
        grid=(B // _GB,),
        in_specs=[pl.BlockSpec((_GB, H), lambda i: (i, 0)),
                  pl.BlockSpec((1, H), lambda i: (0, 0)),
                  pl.BlockSpec((1, H), lambda i: (0, 0)),
                  pl.BlockSpec((H, 1), lambda i: (0, 0)),
                  pl.BlockSpec((_GB, 1), lambda i: (i, 0))],
        out_specs=pl.BlockSpec((_GB, 1), lambda i: (i, 0)),
        out_shape=jax.ShapeDtypeStruct((B, 1), jnp.float32),
    )(z2, a2row, c2row, W3, linf)


def kernel(x, ffm_tables, lin_w, lin_b, bn0_g, bn0_b, W1, b1, g1, bt1,
           W2, b2, g2, bt2, W3, b3):
    xT = x.T.astype(jnp.int32)
    tabsT = jnp.transpose(ffm_tables, (0, 2, 1))
    tabs = _fmt_tables(tabsT).reshape(NF * VPAD, D)
    linw16 = lin_w.reshape(VOCAB // 16, 16)
    pf = jnp.asarray(_PFP)
    pc = jnp.asarray(_PCP)
    toff = jnp.asarray(_TOFF)

    ocross, stats, linsum = _sc_call(xT, tabs, linw16, pf, pc, toff)
    o4 = ocross.reshape(B // 8, KT, 8, 128)

    n0 = float(B * NPAIR)
    s0 = jnp.sum(stats[:, 0, :], axis=0)
    q0 = jnp.sum(stats[:, 1, :], axis=0)
    m0 = s0 / n0
    v0 = jnp.maximum(q0 / n0 - m0 * m0, 0.0)
    a0 = bn0_g * lax.rsqrt(v0 + EPS)
    a0row = jnp.tile(a0, NPS).reshape(1, KT * 128)
    W1p = jnp.pad(W1, ((0, KT * 128 - FFM_OUT), (0, 0)))

    z1, st1 = _mlp1(o4, a0row, W1p)

    m1 = st1[0] / B
    v1 = jnp.maximum(st1[1] / B - m1 * m1, 0.0)
    a1 = g1 * lax.rsqrt(v1 + EPS)
    c1 = bt1 - m1 * a1

    z2, st2 = _mlp2(z1, a1.reshape(1, H), c1.reshape(1, H), W2)

    m2 = st2[0] / B
    v2 = jnp.maximum(st2[1] / B - m2 * m2, 0.0)
    a2 = g2 * lax.rsqrt(v2 + EPS)
    c2 = bt2 - m2 * a2

    linf = (linsum + lin_b[0] + b3[0]).reshape(B, 1)
    out = _mlp3(z2, a2.reshape(1, H), c2.reshape(1, H), W3, linf)
    return out.reshape(B)


# speedup vs baseline: 1.0453x; 1.0453x over previous
import dataclasses
import functools

import numpy as np
import jax
from jax import lax
import jax.numpy as jnp
from jax.experimental import pallas as pl
from jax.experimental.pallas import tpu as pltpu
from jax.experimental.pallas import tpu_sc as plsc

NF = 26
FD = 4000
VOCAB = NF * FD
D = 16
NPAIR = NF * (NF - 1) // 2
B = 4096
H = 400
FFM_OUT = NPAIR * D
EPS = 1e-5

NC, NS = 2, 16
NW = NC * NS
BW = B // NW

VROW = 104448

_II, _JJ = np.triu_indices(NF, k=1)
_PF = np.zeros(2 * NPAIR, np.int32)
_PC = np.zeros(2 * NPAIR, np.int32)
_PF[0::2] = _II
_PC[0::2] = _JJ * VROW + _II * FD
_PF[1::2] = _JJ
_PC[1::2] = _II * VROW + _JJ * FD
NL = 656
_PFP = np.zeros(NL, np.int32)
_PCP = np.zeros(NL, np.int32)
_PFP[: 2 * NPAIR] = _PF
_PCP[: 2 * NPAIR] = _PC

_CHUNKS = [(0, 128), (128, 128), (256, 128), (384, 128), (512, 128), (640, 16)]

_PAIR_UNROLL = 13

KT = 41
NPS = 328
NSC = 336
OROWS = (B // 8) * KT * 64
_TOFF = np.zeros(NSC, np.int32)
_p = np.arange(NPS, dtype=np.int64)
_TOFF[:NPS] = ((_p >> 3) * 64 + (_p & 7)).astype(np.int32)
_TOFF[NPS:] = _TOFF[326]
_SCHUNKS = [(0, 128), (128, 128), (256, 80)]


def _sc_body(xT, tabs, linw16, pf_h, pc_h, toff_h,
             ocross, stats, linsum,
             pf_v, pc_v, toff_v, xblk, lidx0, lidx1, rows0, rows1,
             prod0, prod1, s0a, s1a, s2a, s0b, s1b, s2b,
             li_idx, li_lane, li_rows, li_acc, st_v,
             sem_g0, sem_g1, sem_w0, sem_w1, sem_s):
    c = lax.axis_index("c")
    s = lax.axis_index("s")
    wid = s * NC + c
    base = wid * BW

    pltpu.sync_copy(pf_h, pf_v)
    pltpu.sync_copy(pc_h, pc_v)
    pltpu.sync_copy(toff_h, toff_v)
    pltpu.sync_copy(xT.at[:, pl.ds(base, BW)], xblk)
    iota16 = lax.iota(jnp.int32, 16)

    for i in range(NF):
        for g in range(8):
            off = i * BW + g * 16
            v = xblk[i, pl.ds(g * 16, 16)]
            li_idx[pl.ds(off, 16)] = 250 * i + (v >> 4)
            li_lane[pl.ds(off, 16)] = v & 15
    for i in range(NF):
        pltpu.async_copy(linw16.at[li_idx.at[pl.ds(i * BW, BW)]],
                         li_rows.at[pl.ds(i * BW, BW)], sem_s)
    for i in range(NF):
        pltpu.make_async_copy(linw16.at[li_idx.at[pl.ds(i * BW, BW)]],
                              li_rows.at[pl.ds(i * BW, BW)], sem_s).wait()
    for g in range(8):
        acc = jnp.zeros((16,), jnp.float32)
        for i in range(NF):
            off = i * BW + g * 16
            rvec = off + iota16
            lvec = li_lane[pl.ds(off, 16)]
            acc = acc + plsc.load_gather(li_rows, [rvec, lvec])
        li_acc[pl.ds(g * 16, 16)] = acc
    pltpu.sync_copy(li_acc, linsum.at[pl.ds(base, BW)])

    st_v[0] = jnp.zeros((16,), jnp.float32)
    st_v[1] = jnp.zeros((16,), jnp.float32)

    def build_lidx(b, lidx_ref):
        bsplat = jnp.full((16,), b, jnp.int32)
        for grp in range(NL // 16):
            pfv = pf_v[pl.ds(grp * 16, 16)]
            pcv = pc_v[pl.ds(grp * 16, 16)]
            xv = plsc.load_gather(xblk, [pfv, bsplat])
            lidx_ref[pl.ds(grp * 16, 16)] = xv + pcv

    def fire_gather(lidx_ref, rows_ref, sem):
        for (o, n) in _CHUNKS:
            pltpu.async_copy(tabs.at[lidx_ref.at[pl.ds(o, n)]],
                             rows_ref.at[pl.ds(o, n)], sem)

    def drain_gather(lidx_ref, rows_ref, sem):
        for (o, n) in _CHUNKS:
            pltpu.make_async_copy(tabs.at[lidx_ref.at[pl.ds(o, n)]],
                                  rows_ref.at[pl.ds(o, n)], sem).wait()

    def compute_products(rows_ref, prod_ref):
        sa = jnp.zeros((16,), jnp.float32)
        qa = jnp.zeros((16,), jnp.float32)
        for p in range(NPAIR):
            pr = rows_ref[2 * p] * rows_ref[2 * p + 1]
            prod_ref[p] = pr
            sa = sa + pr
            qa = qa + pr * pr
        st_v[0] = st_v[0] + sa
        st_v[1] = st_v[1] + qa

    def build_sidx(b, srefs):
        b_abs = base + b
        cb = (b_abs >> 3) * (KT * 64) + (b_abs & 7) * 8
        for grp in range(NSC // 16):
            tv = toff_v[pl.ds(grp * 16, 16)]
            val = tv + cb
            if grp < 8:
                srefs[0][pl.ds(grp * 16, 16)] = val
            elif grp < 16:
                srefs[1][pl.ds((grp - 8) * 16, 16)] = val
            else:
                srefs[2][pl.ds((grp - 16) * 16, 16)] = val

    def fire_scatter(prod_ref, srefs, sem):
        for (o, n), sref in zip(_SCHUNKS, srefs):
            pltpu.async_copy(prod_ref.at[pl.ds(o, n)], ocross.at[sref], sem)

    def drain_scatter(prod_ref, srefs, sem):
        for (o, n), sref in zip(_SCHUNKS, srefs):
            pltpu.make_async_copy(prod_ref.at[pl.ds(o, n)], ocross.at[sref],
                                  sem).wait()

    for r in range(NPAIR, NSC):
        prod0[r] = jnp.zeros((16,), jnp.float32)
        prod1[r] = jnp.zeros((16,), jnp.float32)

    build_lidx(0, lidx0)
    fire_gather(lidx0, rows0, sem_g0)
    build_lidx(1, lidx1)
    fire_gather(lidx1, rows1, sem_g1)

    def phase(b, lidx_ref, rows_ref, prod_ref, srefs, sem_g, sem_w):
        drain_gather(lidx_ref, rows_ref, sem_g)

        @pl.when(b >= 2)
        def _():
            drain_scatter(prod_ref, srefs, sem_w)

        build_sidx(b, srefs)
        compute_products(rows_ref, prod_ref)
        fire_scatter(prod_ref, srefs, sem_w)

        @pl.when(b + 2 < BW)
        def _():
            build_lidx(b + 2, lidx_ref)
            fire_gather(lidx_ref, rows_ref, sem_g)

    @pl.loop(0, BW, step=2)
    def _(b):
        phase(b, lidx0, rows0, prod0, (s0a, s1a, s2a), sem_g0, sem_w0)
        phase(b + 1, lidx1, rows1, prod1, (s0b, s1b, s2b), sem_g1, sem_w1)

    drain_scatter(prod0, (s0a, s1a, s2a), sem_w0)
    drain_scatter(prod1, (s0b, s1b, s2b), sem_w1)
    pltpu.sync_copy(st_v, stats.at[wid])


def _sc_call(xT, tabs, linw16, pf, pc, toff):
    mesh = plsc.VectorSubcoreMesh(core_axis_name="c", subcore_axis_name="s")
    cp = pltpu.CompilerParams()
    for fld, val in (("needs_layout_passes", False),
                     ("use_tc_tiling_on_sc", False)):
        if fld in pltpu.CompilerParams.__dataclass_fields__:
            cp = dataclasses.replace(cp, **{fld: val})
    kern = pl.kernel(
        _sc_body,
        compiler_params=cp,
        out_type=(
            jax.ShapeDtypeStruct((OROWS, D), jnp.float32),
            jax.ShapeDtypeStruct((NW, 2, D), jnp.float32),
            jax.ShapeDtypeStruct((B,), jnp.float32),
        ),
        mesh=mesh,
        scratch_types=[
            pltpu.VMEM((NL,), jnp.int32),
            pltpu.VMEM((NL,), jnp.int32),
            pltpu.VMEM((NSC,), jnp.int32),
            pltpu.VMEM((NF, BW), jnp.int32),
            pltpu.VMEM((NL,), jnp.int32),
            pltpu.VMEM((NL,), jnp.int32),
            pltpu.VMEM((NL, D), jnp.float32),
            pltpu.VMEM((NL, D), jnp.float32),
            pltpu.VMEM((NSC, D), jnp.float32),
            pltpu.VMEM((NSC, D), jnp.float32),
            pltpu.VMEM((128,), jnp.int32),
            pltpu.VMEM((128,), jnp.int32),
            pltpu.VMEM((80,), jnp.int32),
            pltpu.VMEM((128,), jnp.int32),
            pltpu.VMEM((128,), jnp.int32),
            pltpu.VMEM((80,), jnp.int32),
            pltpu.VMEM((NF * BW,), jnp.int32),
            pltpu.VMEM((NF * BW,), jnp.int32),
            pltpu.VMEM((NF * BW, D), jnp.float32),
            pltpu.VMEM((BW,), jnp.float32),
            pltpu.VMEM((2, D), jnp.float32),
            pltpu.SemaphoreType.DMA,
            pltpu.SemaphoreType.DMA,
            pltpu.SemaphoreType.DMA,
            pltpu.SemaphoreType.DMA,
            pltpu.SemaphoreType.DMA,
        ],
    )
    return kern(xT, tabs, linw16, pf, pc, toff)


_GB = 256


VPAD = VROW
RPF = VPAD * D // 1024
_VC = 13056


def _fmt_tables(tabsT):
    def body(t_ref, o_ref):
        tr = jnp.transpose(t_ref[0], (1, 0))
        tr3 = tr.reshape(_VC // 8, 8, 16)
        out2 = jnp.concatenate([tr3[:, v0, :] for v0 in range(8)], axis=1)
        o_ref[...] = out2.reshape(_VC // 64, 8, 128)

    return pl.pallas_call(
        body,
        grid=(NF, VPAD // _VC),
        in_specs=[pl.BlockSpec((1, D, _VC), lambda f, c: (f, 0, c))],
        out_specs=pl.BlockSpec((_VC // 64, 8, 128),
                               lambda f, c: (f * (VPAD // _VC) + c, 0, 0)),
        out_shape=jax.ShapeDtypeStruct((NF * RPF, 8, 128), jnp.float32),
    )(tabsT)


def _mlp1(o4, a0row, W1p):
    def body(o_ref, a_ref, w_ref, z_ref, st_ref):
        i = pl.program_id(0)
        z = jnp.zeros((_GB, H), jnp.float32)
        for kt in range(KT):
            piece = o_ref[:, kt].reshape(_GB, 128)
            piece = piece * a_ref[0:1, kt * 128:(kt + 1) * 128]
            z = z + jnp.dot(piece, w_ref[kt * 128:(kt + 1) * 128, :],
                            preferred_element_type=jnp.float32)
        z_ref[...] = z

        @pl.when(i == 0)
        def _():
            st_ref[...] = jnp.zeros_like(st_ref)

        st_ref[0:1, :] += jnp.sum(z, axis=0, keepdims=True)
        st_ref[1:2, :] += jnp.sum(z * z, axis=0, keepdims=True)

    return pl.pallas_call(
        body,
        grid=(B // _GB,),
        in_specs=[pl.BlockSpec((_GB // 8, KT, 8, 128), lambda i: (i, 0, 0, 0)),
                  pl.BlockSpec((1, KT * 128), lambda i: (0, 0)),
                  pl.BlockSpec((KT * 128, H), lambda i: (0, 0))],
        out_specs=[pl.BlockSpec((_GB, H), lambda i: (i, 0)),
                   pl.BlockSpec((2, H), lambda i: (0, 0))],
        out_shape=[jax.ShapeDtypeStruct((B, H), jnp.float32),
                   jax.ShapeDtypeStruct((2, H), jnp.float32)],
    )(o4, a0row, W1p)


def _mlp2(z1, a1row, c1row, W2):
    def body(z_ref, a_ref, c_ref, w_ref, z2_ref, st_ref):
        i = pl.program_id(0)
        h = jnp.maximum(z_ref[...] * a_ref[...] + c_ref[...], 0.0)
        z2 = jnp.dot(h, w_ref[...], preferred_element_type=jnp.float32)
        z2_ref[...] = z2

        @pl.when(i == 0)
        def _():
            st_ref[...] = jnp.zeros_like(st_ref)

        st_ref[0:1, :] += jnp.sum(z2, axis=0, keepdims=True)
        st_ref[1:2, :] += jnp.sum(z2 * z2, axis=0, keepdims=True)

    return pl.pallas_call(
        body,
        grid=(B // _GB,),
        in_specs=[pl.BlockSpec((_GB, H), lambda i: (i, 0)),
                  pl.BlockSpec((1, H), lambda i: (0, 0)),
                  pl.BlockSpec((1, H), lambda i: (0, 0)),
                  pl.BlockSpec((H, H), lambda i: (0, 0))],
        out_specs=[pl.BlockSpec((_GB, H), lambda i: (i, 0)),
                   pl.BlockSpec((2, H), lambda i: (0, 0))],
        out_shape=[jax.ShapeDtypeStruct((B, H), jnp.float32),
                   jax.ShapeDtypeStruct((2, H), jnp.float32)],
    )(z1, a1row, c1row, W2)


def _mlp3(z2, a2row, c2row, W3, linf):
    def body(z_ref, a_ref, c_ref, w_ref, l_ref, o_ref):
        h = jnp.maximum(z_ref[...] * a_ref[...] + c_ref[...], 0.0)
        o = jnp.dot(h, w_ref[...], preferred_element_type=jnp.float32)
        o_ref[...] = jax.nn.sigmoid(o + l_ref[...])

    return pl.pallas_call(
        body,
        grid=(B // _GB,),
        in_specs=[pl.BlockSpec((_GB, H), lambda i: (i, 0)),
                  pl.BlockSpec((1, H), lambda i: (0, 0)),
                  pl.BlockSpec((1, H), lambda i: (0, 0)),
                  pl.BlockSpec((H, 1), lambda i: (0, 0)),
                  pl.BlockSpec((_GB, 1), lambda i: (i, 0))],
        out_specs=pl.BlockSpec((_GB, 1), lambda i: (i, 0)),
        out_shape=jax.ShapeDtypeStruct((B, 1), jnp.float32),
    )(z2, a2row, c2row, W3, linf)


def kernel(x, ffm_tables, lin_w, lin_b, bn0_g, bn0_b, W1, b1, g1, bt1,
           W2, b2, g2, bt2, W3, b3):
    xT = x.T.astype(jnp.int32)
    tabsT = jnp.transpose(ffm_tables, (0, 2, 1))
    tabs = _fmt_tables(tabsT).reshape(NF * VPAD, D)
    linw16 = lin_w.reshape(VOCAB // 16, 16)
    pf = jnp.asarray(_PFP)
    pc = jnp.asarray(_PCP)
    toff = jnp.asarray(_TOFF)

    ocross, stats, linsum = _sc_call(xT, tabs, linw16, pf, pc, toff)
    o4 = ocross.reshape(B // 8, KT, 8, 128)

    n0 = float(B * NPAIR)
    s0 = jnp.sum(stats[:, 0, :], axis=0)
    q0 = jnp.sum(stats[:, 1, :], axis=0)
    m0 = s0 / n0
    v0 = jnp.maximum(q0 / n0 - m0 * m0, 0.0)
    a0 = bn0_g * lax.rsqrt(v0 + EPS)
    a0row = jnp.tile(a0, NPS).reshape(1, KT * 128)
    W1p = jnp.pad(W1, ((0, KT * 128 - FFM_OUT), (0, 0)))

    z1, st1 = _mlp1(o4, a0row, W1p)

    m1 = st1[0] / B
    v1 = jnp.maximum(st1[1] / B - m1 * m1, 0.0)
    a1 = g1 * lax.rsqrt(v1 + EPS)
    c1 = bt1 - m1 * a1

    z2, st2 = _mlp2(z1, a1.reshape(1, H), c1.reshape(1, H), W2)

    m2 = st2[0] / B
    v2 = jnp.maximum(st2[1] / B - m2 * m2, 0.0)
    a2 = g2 * lax.rsqrt(v2 + EPS)
    c2 = bt2 - m2 * a2

    linf = (linsum + lin_b[0] + b3[0]).reshape(B, 1)
    out = _mlp3(z2, a2.reshape(1, H), c2.reshape(1, H), W3, linf)
    return out.reshape(B)
